# Initial kernel scaffold; baseline (speedup 1.0000x reference)
import jax
import jax.numpy as jnp
from jax.experimental import pallas as pl


def kernel(x, V, alpha):
    raise NotImplementedError("write your pallas kernel here")


# speedup vs baseline: 624.5480x; 624.5480x over previous
import math

import jax
import jax.numpy as jnp
from jax.experimental import pallas as pl
from jax.experimental.pallas import tpu as pltpu


def _fc_softmax_kernel(x_ref, vt_ref, a_ref, o_ref, *, k_top):
    a = a_ref[...]
    m = jnp.max(a, axis=1, keepdims=True)
    e = jnp.exp(a - m)
    probs = e / jnp.sum(e, axis=1, keepdims=True)
    s = jnp.clip(k_top * probs, 0.0, 1.0)

    vts = vt_ref[...] * s
    wt = pltpu.roll(vts, 0, 1, stride=1, stride_axis=0)
    o_ref[...] = jax.lax.dot_general(
        x_ref[...], wt,
        dimension_numbers=(((1,), (0,)), ((), ())),
        preferred_element_type=jnp.float32,
        precision=jax.lax.Precision.HIGHEST,
    )


def kernel(x, V, alpha):
    total, diag = V.shape
    batch, in_f = x.shape
    sparsity = 0.1
    k_top = math.ceil(int((1 - sparsity) * in_f * total) / diag)
    return pl.pallas_call(
        lambda x_ref, vt_ref, a_ref, o_ref: _fc_softmax_kernel(
            x_ref, vt_ref, a_ref, o_ref, k_top=float(k_top)),
        out_shape=jax.ShapeDtypeStruct((batch, total), jnp.float32),
    )(x, V.T, alpha.reshape(1, total))


# speedup vs baseline: 1140.0751x; 1.8254x over previous
import math

import jax
import jax.numpy as jnp
from jax.experimental import pallas as pl
from jax.experimental.pallas import tpu as pltpu


def _fc_softmax_kernel(x_ref, v_ref, a_ref, o_ref, *, k_top):
    a = a_ref[...]
    m = jnp.max(a, axis=1, keepdims=True)
    e = jnp.exp(a - m)
    probs = e / jnp.sum(e, axis=1, keepdims=True)
    s = jnp.clip(k_top * probs, 0.0, 1.0)

    vt = v_ref[...].T
    vts = vt * s
    wt = pltpu.roll(vts, 0, 1, stride=1, stride_axis=0)
    o_ref[...] = jax.lax.dot_general(
        x_ref[...], wt,
        dimension_numbers=(((1,), (0,)), ((), ())),
        preferred_element_type=jnp.float32,
        precision=jax.lax.Precision.HIGHEST,
    )


def kernel(x, V, alpha):
    total, diag = V.shape
    batch, in_f = x.shape
    sparsity = 0.1
    k_top = math.ceil(int((1 - sparsity) * in_f * total) / diag)
    return pl.pallas_call(
        lambda x_ref, v_ref, a_ref, o_ref: _fc_softmax_kernel(
            x_ref, v_ref, a_ref, o_ref, k_top=float(k_top)),
        out_shape=jax.ShapeDtypeStruct((batch, total), jnp.float32),
    )(x, V, alpha.reshape(1, total))
